# baseline (speedup 1.0000x reference)
import functools

import jax
import jax.numpy as jnp
from jax import lax
from jax.experimental import pallas as pl
from jax.experimental.pallas import tpu as pltpu
from jax.experimental.pallas import tpu_sc as plsc

N = 10000
N_PAD = 10240
E = 320000
D_IN = 128
D_OUT = 64
NTILES = 16
NCORES = 2
ROWS = N_PAD // NTILES
RB = 64
NRB = ROWS // RB
CHUNK = 128
NCHUNKS = 160
E_PAD = NTILES * NCHUNKS * CHUNK
EPT = E_PAD // NTILES
NBUF = 8
NGRP = NCHUNKS // NBUF

_mesh = plsc.VectorSubcoreMesh(core_axis_name="c", subcore_axis_name="s")


def _rsqrt16(x):
    y = jnp.full((16,), 0.5, jnp.float32)
    for k in range(1, 10):
        y = jnp.where(x < jnp.float32(4.0 ** k), y,
                      jnp.full((16,), 2.0 ** (-k - 1), jnp.float32))
    for _ in range(6):
        y = y * (1.5 - 0.5 * x * y * y)
    return y


def _fill16(ref, nrows, value):
    v = jnp.full((16,), value, jnp.float32)

    def body(i, _):
        ref[i] = v
        return 0

    lax.fori_loop(0, nrows, body, 0)


def _scale_rows(rowb, dinvb, blk, ncol16, square, bias=None):
    def body(i, _):
        dvv = plsc.load_gather(
            dinvb, [jnp.full((16,), blk * RB + i, jnp.int32)])
        if square:
            dvv = dvv * dvv
        for j in range(ncol16):
            v = rowb[i, pl.ds(j * 16, 16)] * dvv
            if bias is not None:
                v = v + bias[j]
            rowb[i, pl.ds(j * 16, 16)] = v
        return 0

    lax.fori_loop(0, RB, body, 0)


def _edge_pass_staged(srcm, dstm, c, s, sib, dib, cur, acc, gbufs, gsems,
                      ssems):
    nbuf = len(gbufs)
    pltpu.sync_copy(srcm.at[c, s, pl.ds(0, nbuf)], sib.at[0])
    pltpu.sync_copy(dstm.at[s, pl.ds(0, nbuf)], dib.at[0])
    for b in range(nbuf):
        pltpu.async_copy(cur.at[sib.at[0, b]], gbufs[b], gsems[b])

    def grp(g, _):
        par = lax.rem(g, 2)
        nxt = 1 - par
        pltpu.sync_copy(srcm.at[c, s, pl.ds((g + 1) * nbuf, nbuf)],
                        sib.at[nxt])
        pltpu.sync_copy(dstm.at[s, pl.ds((g + 1) * nbuf, nbuf)],
                        dib.at[nxt])
        for b in range(nbuf):
            pltpu.make_async_copy(cur.at[sib.at[par, b]], gbufs[b],
                                  gsems[b]).wait()
            pltpu.async_copy(gbufs[b], acc.at[dib.at[par, b]], ssems[b],
                             add=True)
        for b in range(nbuf):
            pltpu.make_async_copy(gbufs[b], acc.at[dib.at[par, b]],
                                  ssems[b]).wait()
            pltpu.async_copy(cur.at[sib.at[nxt, b]], gbufs[b], gsems[b])
        return 0

    lax.fori_loop(0, NGRP - 1, grp, 0)
    lpar = (NGRP - 1) % 2
    for b in range(nbuf):
        pltpu.make_async_copy(cur.at[sib.at[lpar, b]], gbufs[b],
                              gsems[b]).wait()
        pltpu.async_copy(gbufs[b], acc.at[dib.at[lpar, b]], ssems[b],
                         add=True)
    for b in range(nbuf):
        pltpu.make_async_copy(gbufs[b], acc.at[dib.at[lpar, b]],
                              ssems[b]).wait()


def _k1_body(feat, srcm, dstm, out_p, out_dinv, cur,
             acc, deg, g0, g1, g2, g3, g4, g5, g6, g7, rowb, degb, onesb,
             sib, dib, dinvb,
             gs0, gs1, gs2, gs3, gs4, gs5, gs6, gs7,
             ss0, ss1, ss2, ss3, ss4, ss5, ss6, ss7):
    c = lax.axis_index("c")
    s = lax.axis_index("s")
    r0 = s * ROWS
    cr0 = c * N_PAD + r0
    c64 = pl.multiple_of(c * 64, 64)
    gbufs = (g0, g1, g2, g3, g4, g5, g6, g7)
    gsems = (gs0, gs1, gs2, gs3, gs4, gs5, gs6, gs7)
    ssems = (ss0, ss1, ss2, ss3, ss4, ss5, ss6, ss7)

    _fill16(degb, RB, 0.0)
    for blk in range(NRB):
        pltpu.sync_copy(degb, deg.at[pl.ds(r0 + blk * RB, RB)])
    _fill16(onesb, CHUNK, 1.0)
    plsc.subcore_barrier()

    pltpu.sync_copy(dstm.at[s, pl.ds(0, NBUF)], dib.at[0])
    for b in range(NBUF):
        pltpu.async_copy(onesb, deg.at[dib.at[0, b]], ssems[b], add=True)

    def deg_grp(g, _):
        par = lax.rem(g, 2)
        nxt = 1 - par
        pltpu.sync_copy(dstm.at[s, pl.ds((g + 1) * NBUF, NBUF)],
                        dib.at[nxt])
        for b in range(NBUF):
            pltpu.make_async_copy(onesb, deg.at[dib.at[par, b]],
                                  ssems[b]).wait()
            pltpu.async_copy(onesb, deg.at[dib.at[nxt, b]], ssems[b],
                             add=True)
        return 0

    lax.fori_loop(0, NGRP - 1, deg_grp, 0)
    dpar = (NGRP - 1) % 2
    for b in range(NBUF):
        pltpu.make_async_copy(onesb, deg.at[dib.at[dpar, b]],
                              ssems[b]).wait()
    plsc.subcore_barrier()

    lanes = lax.iota(jnp.int32, 16)
    zeros16 = jnp.zeros((16,), jnp.int32)
    for blk in range(NRB):
        pltpu.sync_copy(deg.at[pl.ds(r0 + blk * RB, RB)], degb)
        for q in range(RB // 16):
            v = plsc.load_gather(degb, [lanes + q * 16, zeros16])
            dinvb[pl.ds(blk * RB + q * 16, 16)] = _rsqrt16(v + 1.0)

    @pl.when(c == 0)
    def _():
        pltpu.sync_copy(dinvb, out_dinv.at[pl.ds(r0, ROWS)])

    for blk in range(NRB):
        rb0 = r0 + blk * RB
        pltpu.sync_copy(feat.at[pl.ds(rb0, RB), pl.ds(c64, 64)], rowb)
        _scale_rows(rowb, dinvb, blk, 4, square=False)
        pltpu.sync_copy(rowb, cur.at[pl.ds(cr0 + blk * RB, RB)])
        pltpu.sync_copy(rowb, acc.at[pl.ds(rb0, RB)])
    plsc.subcore_barrier()

    _edge_pass_staged(srcm, dstm, c, s, sib, dib, cur, acc, gbufs, gsems,
                      ssems)
    plsc.subcore_barrier()

    for blk in range(NRB):
        rb0 = r0 + blk * RB
        pltpu.sync_copy(acc.at[pl.ds(rb0, RB)], rowb)
        _scale_rows(rowb, dinvb, blk, 4, square=True)
        pltpu.sync_copy(rowb, cur.at[pl.ds(cr0 + blk * RB, RB)])
        pltpu.sync_copy(rowb, acc.at[pl.ds(rb0, RB)])
    plsc.subcore_barrier()

    _edge_pass_staged(srcm, dstm, c, s, sib, dib, cur, acc, gbufs, gsems,
                      ssems)
    plsc.subcore_barrier()

    for blk in range(NRB):
        rb0 = r0 + blk * RB
        pltpu.sync_copy(acc.at[pl.ds(rb0, RB)], rowb)
        pltpu.sync_copy(rowb, out_p.at[c, pl.ds(rb0, RB)])


_k1 = functools.partial(
    pl.kernel,
    out_type=[
        jax.ShapeDtypeStruct((NCORES, N_PAD, 64), jnp.float32),
        jax.ShapeDtypeStruct((N_PAD,), jnp.float32),
        jax.ShapeDtypeStruct((NCORES * N_PAD, 64), jnp.float32),
    ],
    mesh=_mesh,
    compiler_params=pltpu.CompilerParams(use_tc_tiling_on_sc=False, needs_layout_passes=False),
    scratch_types=[
        pltpu.VMEM_SHARED((N_PAD, 64), jnp.float32),
        pltpu.VMEM_SHARED((N_PAD, 16), jnp.float32),
    ] + [pltpu.VMEM((CHUNK, 64), jnp.float32)] * 8 + [
        pltpu.VMEM((RB, 64), jnp.float32),
        pltpu.VMEM((RB, 16), jnp.float32),
        pltpu.VMEM((CHUNK, 16), jnp.float32),
        pltpu.VMEM((2, NBUF, CHUNK), jnp.int32),
        pltpu.VMEM((2, NBUF, CHUNK), jnp.int32),
        pltpu.VMEM((ROWS,), jnp.float32),
    ] + [pltpu.SemaphoreType.DMA] * 16,
    name="sc_prop_layer1",
)(_k1_body)


def _k3_body(mt, srcm, dstm, dinv_in, b2_in, out, curw,
             acc, srcv, dstv, g0, g1, g2, g3, g4, g5, g6, g7, rowb,
             dinvb, b2v,
             gs0, gs1, gs2, gs3, gs4, gs5, gs6, gs7,
             ss0, ss1, ss2, ss3, ss4, ss5, ss6, ss7):
    c = lax.axis_index("c")
    s = lax.axis_index("s")
    r0 = s * ROWS
    col0 = pl.multiple_of(c * 32, 32)
    cr0 = c * N_PAD + r0
    gbufs = (g0, g1, g2, g3, g4, g5, g6, g7)
    gsems = (gs0, gs1, gs2, gs3, gs4, gs5, gs6, gs7)
    ssems = (ss0, ss1, ss2, ss3, ss4, ss5, ss6, ss7)

    pltpu.sync_copy(srcm.at[c, s], srcv)
    pltpu.sync_copy(dstm.at[s], dstv)
    pltpu.sync_copy(dinv_in.at[pl.ds(r0, ROWS)], dinvb)
    pltpu.sync_copy(b2_in, b2v)

    for blk in range(NRB):
        rb0 = r0 + blk * RB
        pltpu.sync_copy(mt.at[pl.ds(cr0 + blk * RB, RB)], rowb)
        pltpu.sync_copy(rowb, acc.at[pl.ds(rb0, RB)])
    plsc.subcore_barrier()

    def edge_ring(cur):
        for b in range(NBUF):
            pltpu.async_copy(cur.at[srcv.at[b]], gbufs[b], gsems[b])

        def grp(g, _):
            base = g * NBUF
            for b in range(NBUF):
                pltpu.make_async_copy(cur.at[srcv.at[base + b]], gbufs[b],
                                      gsems[b]).wait()
                pltpu.async_copy(gbufs[b], acc.at[dstv.at[base + b]],
                                 ssems[b], add=True)
            for b in range(NBUF):
                pltpu.make_async_copy(gbufs[b], acc.at[dstv.at[base + b]],
                                      ssems[b]).wait()
                pltpu.async_copy(cur.at[srcv.at[base + NBUF + b]],
                                 gbufs[b], gsems[b])
            return 0

        lax.fori_loop(0, NGRP - 1, grp, 0)
        base = (NGRP - 1) * NBUF
        for b in range(NBUF):
            pltpu.make_async_copy(cur.at[srcv.at[base + b]], gbufs[b],
                                  gsems[b]).wait()
            pltpu.async_copy(gbufs[b], acc.at[dstv.at[base + b]],
                             ssems[b], add=True)
        for b in range(NBUF):
            pltpu.make_async_copy(gbufs[b], acc.at[dstv.at[base + b]],
                                  ssems[b]).wait()

    edge_ring(mt)
    plsc.subcore_barrier()

    for blk in range(NRB):
        rb0 = r0 + blk * RB
        pltpu.sync_copy(acc.at[pl.ds(rb0, RB)], rowb)
        _scale_rows(rowb, dinvb, blk, 2, square=True)
        pltpu.sync_copy(rowb, curw.at[pl.ds(cr0 + blk * RB, RB)])
        pltpu.sync_copy(rowb, acc.at[pl.ds(rb0, RB)])
    plsc.subcore_barrier()

    edge_ring(curw)
    plsc.subcore_barrier()

    b2c = [b2v[pl.ds(col0 + j * 16, 16)] for j in range(2)]
    for blk in range(NRB):
        rb0 = r0 + blk * RB
        pltpu.sync_copy(acc.at[pl.ds(rb0, RB)], rowb)
        _scale_rows(rowb, dinvb, blk, 2, square=False, bias=b2c)

        @pl.when(rb0 + RB <= N)
        def _():
            pltpu.sync_copy(rowb, out.at[pl.ds(rb0, RB), pl.ds(col0, 32)])

        @pl.when((rb0 < N) & (rb0 + RB > N))
        def _():
            pltpu.sync_copy(rowb.at[pl.ds(0, N % RB)],
                            out.at[pl.ds(rb0, N % RB), pl.ds(col0, 32)])


_k3 = functools.partial(
    pl.kernel,
    out_type=[
        jax.ShapeDtypeStruct((N, D_OUT), jnp.float32),
        jax.ShapeDtypeStruct((NCORES * N_PAD, 32), jnp.float32),
    ],
    mesh=_mesh,
    compiler_params=pltpu.CompilerParams(use_tc_tiling_on_sc=False, needs_layout_passes=False),
    scratch_types=[
        pltpu.VMEM_SHARED((N_PAD, 32), jnp.float32),
        pltpu.VMEM((NCHUNKS, CHUNK), jnp.int32),
        pltpu.VMEM((NCHUNKS, CHUNK), jnp.int32),
    ] + [pltpu.VMEM((CHUNK, 32), jnp.float32)] * 8 + [
        pltpu.VMEM((RB, 32), jnp.float32),
        pltpu.VMEM((ROWS,), jnp.float32),
        pltpu.VMEM((D_OUT,), jnp.float32),
    ] + [pltpu.SemaphoreType.DMA] * 16,
    name="sc_prop_layer2",
)(_k3_body)


def _tc_body(p_ref, dinv_ref, w1_ref, b1_ref, w2_ref, out_ref):
    dv = dinv_ref[...]
    w1 = w1_ref[...]
    h = (jnp.dot(p_ref[0] * dv, w1[:64, :], preferred_element_type=jnp.float32,
                 precision=lax.Precision.HIGHEST)
         + jnp.dot(p_ref[1] * dv, w1[64:, :],
                   preferred_element_type=jnp.float32,
                   precision=lax.Precision.HIGHEST))
    h = jnp.maximum(h + b1_ref[...], 0.0)
    m = jnp.dot(h, w2_ref[...], preferred_element_type=jnp.float32,
                precision=lax.Precision.HIGHEST)
    m = m * dv
    out_ref[0] = m[:, :32]
    out_ref[1] = m[:, 32:]


_TC_BLK = 1024


def _tc_dense(p, dinv2d, W1, b1, W2):
    grid = (N_PAD // _TC_BLK,)
    return pl.pallas_call(
        _tc_body,
        grid=grid,
        in_specs=[
            pl.BlockSpec((NCORES, _TC_BLK, 64), lambda i: (0, i, 0)),
            pl.BlockSpec((_TC_BLK, 1), lambda i: (i, 0)),
            pl.BlockSpec((D_IN, D_IN), lambda i: (0, 0)),
            pl.BlockSpec((1, D_IN), lambda i: (0, 0)),
            pl.BlockSpec((D_IN, D_OUT), lambda i: (0, 0)),
        ],
        out_specs=pl.BlockSpec((NCORES, _TC_BLK, 32), lambda i: (0, i, 0)),
        out_shape=jax.ShapeDtypeStruct((NCORES, N_PAD, 32), jnp.float32),
        name="tc_dense_mid",
    )(p, dinv2d, W1, b1, W2)


def kernel(features, edge_index, W1, b1, W2, b2):
    sink = jnp.full((E_PAD - E,), N_PAD - 1, jnp.int32)
    src = jnp.concatenate([edge_index[0].astype(jnp.int32), sink])
    dst = jnp.concatenate([edge_index[1].astype(jnp.int32), sink])
    src = src.reshape(NTILES, NCHUNKS, CHUNK)
    dst = dst.reshape(NTILES, NCHUNKS, CHUNK)
    src2 = jnp.stack([src, src + N_PAD])
    feat_p = jnp.pad(features, ((0, N_PAD - N), (0, 0)))
    p, dinv, _ = _k1(feat_p, src2, dst)
    mt = _tc_dense(p, dinv.reshape(N_PAD, 1), W1, b1.reshape(1, D_IN), W2)
    out, _ = _k3(mt.reshape(NCORES * N_PAD, 32), src2, dst, dinv, b2)
    return out


# speedup vs baseline: 2.1963x; 2.1963x over previous
import functools

import jax
import jax.numpy as jnp
from jax import lax
from jax.experimental import pallas as pl
from jax.experimental.pallas import tpu as pltpu
from jax.experimental.pallas import tpu_sc as plsc

N = 10000
N_PAD = 10240
E = 320000
D_IN = 128
D_OUT = 64
NTILES = 16
NCORES = 2
ROWS = N_PAD // NTILES
RB = 128
NRB = ROWS // RB
CHUNK = 100
EPT = E // NTILES
NCHUNKS = EPT // CHUNK
NBUF = 8
NGRP = NCHUNKS // NBUF

_mesh = plsc.VectorSubcoreMesh(core_axis_name="c", subcore_axis_name="s")


def _rsqrt16(x):
    y = jnp.full((16,), 0.5, jnp.float32)
    for k in range(1, 10):
        y = jnp.where(x < jnp.float32(4.0 ** k), y,
                      jnp.full((16,), 2.0 ** (-k - 1), jnp.float32))
    for _ in range(6):
        y = y * (1.5 - 0.5 * x * y * y)
    return y


def _fill16(ref, nrows, value):
    v = jnp.full((16,), value, jnp.float32)

    def body(i, _):
        ref[i] = v
        return 0

    lax.fori_loop(0, nrows, body, 0)


def _scale_rows(rowb, dinvb, blk, ncol16, square, bias=None):
    def body(i, _):
        dvv = plsc.load_gather(
            dinvb, [jnp.full((16,), blk * RB + i, jnp.int32)])
        if square:
            dvv = dvv * dvv
        for j in range(ncol16):
            v = rowb[i, pl.ds(j * 16, 16)] * dvv
            if bias is not None:
                v = v + bias[j]
            rowb[i, pl.ds(j * 16, 16)] = v
        return 0

    lax.fori_loop(0, RB, body, 0)


def _edge_pass_staged(srcm, dstm, c, s, sib, dib, cur, acc, gbufs, gsems,
                      ssems):
    nbuf = len(gbufs)
    pltpu.sync_copy(srcm.at[c, s, pl.ds(0, nbuf)], sib.at[0])
    pltpu.sync_copy(dstm.at[s, pl.ds(0, nbuf)], dib.at[0])
    for b in range(nbuf):
        pltpu.async_copy(cur.at[sib.at[0, b]], gbufs[b], gsems[b])

    def grp(g, _):
        par = lax.rem(g, 2)
        nxt = 1 - par
        pltpu.sync_copy(srcm.at[c, s, pl.ds((g + 1) * nbuf, nbuf)],
                        sib.at[nxt])
        pltpu.sync_copy(dstm.at[s, pl.ds((g + 1) * nbuf, nbuf)],
                        dib.at[nxt])
        for b in range(nbuf):
            pltpu.make_async_copy(cur.at[sib.at[par, b]], gbufs[b],
                                  gsems[b]).wait()
            pltpu.async_copy(gbufs[b], acc.at[dib.at[par, b]], ssems[b],
                             add=True)
        for b in range(nbuf):
            pltpu.make_async_copy(gbufs[b], acc.at[dib.at[par, b]],
                                  ssems[b]).wait()
            pltpu.async_copy(cur.at[sib.at[nxt, b]], gbufs[b], gsems[b])
        return 0

    lax.fori_loop(0, NGRP - 1, grp, 0)
    lpar = (NGRP - 1) % 2
    for b in range(nbuf):
        pltpu.make_async_copy(cur.at[sib.at[lpar, b]], gbufs[b],
                              gsems[b]).wait()
        pltpu.async_copy(gbufs[b], acc.at[dib.at[lpar, b]], ssems[b],
                         add=True)
    for b in range(nbuf):
        pltpu.make_async_copy(gbufs[b], acc.at[dib.at[lpar, b]],
                              ssems[b]).wait()


def _k1_body(feat, srcm, dstm, out_p, out_dinv, cur,
             acc, deg, g0, g1, g2, g3, g4, g5, g6, g7, rowb, degb, onesb,
             sib, dib, dinvb,
             gs0, gs1, gs2, gs3, gs4, gs5, gs6, gs7,
             ss0, ss1, ss2, ss3, ss4, ss5, ss6, ss7):
    c = lax.axis_index("c")
    s = lax.axis_index("s")
    r0 = s * ROWS
    cr0 = c * N_PAD + r0
    c64 = pl.multiple_of(c * 64, 64)
    gbufs = (g0, g1, g2, g3, g4, g5, g6, g7)
    gsems = (gs0, gs1, gs2, gs3, gs4, gs5, gs6, gs7)
    ssems = (ss0, ss1, ss2, ss3, ss4, ss5, ss6, ss7)

    _fill16(degb, RB, 0.0)
    for blk in range(NRB):
        pltpu.sync_copy(degb, deg.at[pl.ds(r0 + blk * RB, RB)])
    _fill16(onesb, CHUNK, 1.0)
    plsc.subcore_barrier()

    pltpu.sync_copy(dstm.at[s, pl.ds(0, NBUF)], dib.at[0])
    for b in range(NBUF):
        pltpu.async_copy(onesb, deg.at[dib.at[0, b]], ssems[b], add=True)

    def deg_grp(g, _):
        par = lax.rem(g, 2)
        nxt = 1 - par
        pltpu.sync_copy(dstm.at[s, pl.ds((g + 1) * NBUF, NBUF)],
                        dib.at[nxt])
        for b in range(NBUF):
            pltpu.make_async_copy(onesb, deg.at[dib.at[par, b]],
                                  ssems[b]).wait()
            pltpu.async_copy(onesb, deg.at[dib.at[nxt, b]], ssems[b],
                             add=True)
        return 0

    lax.fori_loop(0, NGRP - 1, deg_grp, 0)
    dpar = (NGRP - 1) % 2
    for b in range(NBUF):
        pltpu.make_async_copy(onesb, deg.at[dib.at[dpar, b]],
                              ssems[b]).wait()
    plsc.subcore_barrier()

    lanes = lax.iota(jnp.int32, 16)
    zeros16 = jnp.zeros((16,), jnp.int32)
    for blk in range(NRB):
        pltpu.sync_copy(deg.at[pl.ds(r0 + blk * RB, RB)], degb)
        for q in range(RB // 16):
            v = plsc.load_gather(degb, [lanes + q * 16, zeros16])
            dinvb[pl.ds(blk * RB + q * 16, 16)] = _rsqrt16(v + 1.0)

    @pl.when(c == 0)
    def _():
        pltpu.sync_copy(dinvb, out_dinv.at[pl.ds(r0, ROWS)])

    for blk in range(NRB):
        rb0 = r0 + blk * RB
        pltpu.sync_copy(feat.at[pl.ds(rb0, RB), pl.ds(c64, 64)], rowb)
        _scale_rows(rowb, dinvb, blk, 4, square=False)
        pltpu.sync_copy(rowb, cur.at[pl.ds(cr0 + blk * RB, RB)])
        pltpu.sync_copy(rowb, acc.at[pl.ds(rb0, RB)])
    plsc.subcore_barrier()

    _edge_pass_staged(srcm, dstm, c, s, sib, dib, cur, acc, gbufs, gsems,
                      ssems)
    plsc.subcore_barrier()

    for blk in range(NRB):
        rb0 = r0 + blk * RB
        pltpu.sync_copy(acc.at[pl.ds(rb0, RB)], rowb)
        _scale_rows(rowb, dinvb, blk, 4, square=True)
        pltpu.sync_copy(rowb, cur.at[pl.ds(cr0 + blk * RB, RB)])
        pltpu.sync_copy(rowb, acc.at[pl.ds(rb0, RB)])
    plsc.subcore_barrier()

    _edge_pass_staged(srcm, dstm, c, s, sib, dib, cur, acc, gbufs, gsems,
                      ssems)
    plsc.subcore_barrier()

    for blk in range(NRB):
        rb0 = r0 + blk * RB
        pltpu.sync_copy(acc.at[pl.ds(rb0, RB)], rowb)
        pltpu.sync_copy(rowb, out_p.at[c, pl.ds(rb0, RB)])


_k1 = functools.partial(
    pl.kernel,
    out_type=[
        jax.ShapeDtypeStruct((NCORES, N_PAD, 64), jnp.float32),
        jax.ShapeDtypeStruct((N_PAD,), jnp.float32),
        jax.ShapeDtypeStruct((NCORES * N_PAD, 64), jnp.float32),
    ],
    mesh=_mesh,
    compiler_params=pltpu.CompilerParams(use_tc_tiling_on_sc=False, needs_layout_passes=False),
    scratch_types=[
        pltpu.VMEM_SHARED((N_PAD, 64), jnp.float32),
        pltpu.VMEM_SHARED((N_PAD, 16), jnp.float32),
    ] + [pltpu.VMEM((CHUNK, 64), jnp.float32)] * 8 + [
        pltpu.VMEM((RB, 64), jnp.float32),
        pltpu.VMEM((RB, 16), jnp.float32),
        pltpu.VMEM((CHUNK, 16), jnp.float32),
        pltpu.VMEM((2, NBUF, CHUNK), jnp.int32),
        pltpu.VMEM((2, NBUF, CHUNK), jnp.int32),
        pltpu.VMEM((ROWS,), jnp.float32),
    ] + [pltpu.SemaphoreType.DMA] * 16,
    name="sc_prop_layer1",
)(_k1_body)


def _k3_body(mt, srcm, dstm, dinv_in, b2_in, out, curw,
             acc, srcv, dstv, g0, g1, g2, g3, g4, g5, g6, g7, rowb,
             dinvb, b2v,
             gs0, gs1, gs2, gs3, gs4, gs5, gs6, gs7,
             ss0, ss1, ss2, ss3, ss4, ss5, ss6, ss7):
    c = lax.axis_index("c")
    s = lax.axis_index("s")
    r0 = s * ROWS
    col0 = pl.multiple_of(c * 32, 32)
    cr0 = c * N_PAD + r0
    gbufs = (g0, g1, g2, g3, g4, g5, g6, g7)
    gsems = (gs0, gs1, gs2, gs3, gs4, gs5, gs6, gs7)
    ssems = (ss0, ss1, ss2, ss3, ss4, ss5, ss6, ss7)

    pltpu.sync_copy(srcm.at[c, s], srcv)
    pltpu.sync_copy(dstm.at[s], dstv)
    pltpu.sync_copy(dinv_in.at[pl.ds(r0, ROWS)], dinvb)
    pltpu.sync_copy(b2_in, b2v)

    for blk in range(NRB):
        rb0 = r0 + blk * RB
        pltpu.sync_copy(mt.at[pl.ds(cr0 + blk * RB, RB)], rowb)
        pltpu.sync_copy(rowb, acc.at[pl.ds(rb0, RB)])
    plsc.subcore_barrier()

    def edge_ring(cur):
        for b in range(NBUF):
            pltpu.async_copy(cur.at[srcv.at[b]], gbufs[b], gsems[b])

        def grp(g, _):
            base = g * NBUF
            for b in range(NBUF):
                pltpu.make_async_copy(cur.at[srcv.at[base + b]], gbufs[b],
                                      gsems[b]).wait()
                pltpu.async_copy(gbufs[b], acc.at[dstv.at[base + b]],
                                 ssems[b], add=True)
            for b in range(NBUF):
                pltpu.make_async_copy(gbufs[b], acc.at[dstv.at[base + b]],
                                      ssems[b]).wait()
                pltpu.async_copy(cur.at[srcv.at[base + NBUF + b]],
                                 gbufs[b], gsems[b])
            return 0

        lax.fori_loop(0, NGRP - 1, grp, 0)
        base = (NGRP - 1) * NBUF
        for b in range(NBUF):
            pltpu.make_async_copy(cur.at[srcv.at[base + b]], gbufs[b],
                                  gsems[b]).wait()
            pltpu.async_copy(gbufs[b], acc.at[dstv.at[base + b]],
                             ssems[b], add=True)
        for b in range(NBUF):
            pltpu.make_async_copy(gbufs[b], acc.at[dstv.at[base + b]],
                                  ssems[b]).wait()

    edge_ring(mt)
    plsc.subcore_barrier()

    for blk in range(NRB):
        rb0 = r0 + blk * RB
        pltpu.sync_copy(acc.at[pl.ds(rb0, RB)], rowb)
        _scale_rows(rowb, dinvb, blk, 2, square=True)
        pltpu.sync_copy(rowb, curw.at[pl.ds(cr0 + blk * RB, RB)])
        pltpu.sync_copy(rowb, acc.at[pl.ds(rb0, RB)])
    plsc.subcore_barrier()

    edge_ring(curw)
    plsc.subcore_barrier()

    b2c = [b2v[pl.ds(col0 + j * 16, 16)] for j in range(2)]
    for blk in range(NRB):
        rb0 = r0 + blk * RB
        pltpu.sync_copy(acc.at[pl.ds(rb0, RB)], rowb)
        _scale_rows(rowb, dinvb, blk, 2, square=False, bias=b2c)

        @pl.when(rb0 + RB <= N)
        def _():
            pltpu.sync_copy(rowb, out.at[pl.ds(rb0, RB), pl.ds(col0, 32)])

        @pl.when((rb0 < N) & (rb0 + RB > N))
        def _():
            pltpu.sync_copy(rowb.at[pl.ds(0, N % RB)],
                            out.at[pl.ds(rb0, N % RB), pl.ds(col0, 32)])


_k3 = functools.partial(
    pl.kernel,
    out_type=[
        jax.ShapeDtypeStruct((N, D_OUT), jnp.float32),
        jax.ShapeDtypeStruct((NCORES * N_PAD, 32), jnp.float32),
    ],
    mesh=_mesh,
    compiler_params=pltpu.CompilerParams(use_tc_tiling_on_sc=False, needs_layout_passes=False),
    scratch_types=[
        pltpu.VMEM_SHARED((N_PAD, 32), jnp.float32),
        pltpu.VMEM((NCHUNKS, CHUNK), jnp.int32),
        pltpu.VMEM((NCHUNKS, CHUNK), jnp.int32),
    ] + [pltpu.VMEM((CHUNK, 32), jnp.float32)] * 8 + [
        pltpu.VMEM((RB, 32), jnp.float32),
        pltpu.VMEM((ROWS,), jnp.float32),
        pltpu.VMEM((D_OUT,), jnp.float32),
    ] + [pltpu.SemaphoreType.DMA] * 16,
    name="sc_prop_layer2",
)(_k3_body)


def _tc_body(p_ref, dinv_ref, w1_ref, b1_ref, w2_ref, out_ref):
    dv = dinv_ref[...]
    w1 = w1_ref[...]
    h = (jnp.dot(p_ref[0] * dv, w1[:64, :], preferred_element_type=jnp.float32,
                 precision=lax.Precision.HIGHEST)
         + jnp.dot(p_ref[1] * dv, w1[64:, :],
                   preferred_element_type=jnp.float32,
                   precision=lax.Precision.HIGHEST))
    h = jnp.maximum(h + b1_ref[...], 0.0)
    m = jnp.dot(h, w2_ref[...], preferred_element_type=jnp.float32,
                precision=lax.Precision.HIGHEST)
    m = m * dv
    out_ref[0] = m[:, :32]
    out_ref[1] = m[:, 32:]


_TC_BLK = 1024


def _tc_dense(p, dinv2d, W1, b1, W2):
    grid = (N_PAD // _TC_BLK,)
    return pl.pallas_call(
        _tc_body,
        grid=grid,
        in_specs=[
            pl.BlockSpec((NCORES, _TC_BLK, 64), lambda i: (0, i, 0)),
            pl.BlockSpec((_TC_BLK, 1), lambda i: (i, 0)),
            pl.BlockSpec((D_IN, D_IN), lambda i: (0, 0)),
            pl.BlockSpec((1, D_IN), lambda i: (0, 0)),
            pl.BlockSpec((D_IN, D_OUT), lambda i: (0, 0)),
        ],
        out_specs=pl.BlockSpec((NCORES, _TC_BLK, 32), lambda i: (0, i, 0)),
        out_shape=jax.ShapeDtypeStruct((NCORES, N_PAD, 32), jnp.float32),
        name="tc_dense_mid",
    )(p, dinv2d, W1, b1, W2)


def kernel(features, edge_index, W1, b1, W2, b2):
    src = edge_index[0].astype(jnp.int32).reshape(NTILES, NCHUNKS, CHUNK)
    dst = edge_index[1].astype(jnp.int32).reshape(NTILES, NCHUNKS, CHUNK)
    src2 = jnp.stack([src, src + N_PAD])
    feat_p = jnp.pad(features, ((0, N_PAD - N), (0, 0)))
    p, dinv, _ = _k1(feat_p, src2, dst)
    mt = _tc_dense(p, dinv.reshape(N_PAD, 1), W1, b1.reshape(1, D_IN), W2)
    out, _ = _k3(mt.reshape(NCORES * N_PAD, 32), src2, dst, dinv, b2)
    return out
